# speedup vs baseline: 1.1695x; 1.1695x over previous
import functools

import jax
import jax.numpy as jnp
from jax import lax
from jax.experimental import pallas as pl
from jax.experimental.pallas import tpu as pltpu
from jax.experimental.pallas import tpu_sc as plsc

E = 8
K = 2
H = 1024
I = 3584
T = 2048
P = T * K
EP = 128

TM = 128
NT = P // TM + E
NS = NT * TM
NI = 2
TI = I // NI

_NW = 32


def _router(x, gate_w_pad):
    def body(x_ref, gw_ref, vals_ref, idx_ref):
        logits = jnp.dot(x_ref[...], gw_ref[...],
                         preferred_element_type=jnp.float32)
        cols = lax.broadcasted_iota(jnp.int32, (T, EP), 1)
        lm = jnp.where(cols < E, logits, -1e30)
        l1 = jnp.max(lm, axis=1, keepdims=True)
        i1 = jnp.min(jnp.where(lm == l1, cols, EP), axis=1, keepdims=True)
        lm2 = jnp.where(cols == i1, -1e30, lm)
        l2 = jnp.max(lm2, axis=1, keepdims=True)
        i2 = jnp.min(jnp.where(lm2 == l2, cols, EP), axis=1, keepdims=True)
        d = jnp.exp(l2 - l1)
        v1 = 1.0 / (1.0 + d)
        v2 = 1.0 - v1
        vals_ref[...] = jnp.concatenate([v1, v2], axis=1)
        idx_ref[...] = jnp.concatenate([i1, i2], axis=1)

    return pl.pallas_call(
        body,
        out_shape=(jax.ShapeDtypeStruct((T, K), jnp.float32),
                   jax.ShapeDtypeStruct((T, K), jnp.int32)),
    )(x, gate_w_pad)


def _dispatch_meta(topk_vals, topk_idx):
    e_flat = topk_idx.reshape(-1)
    oh = (e_flat[:, None] == jnp.arange(E)[None, :]).astype(jnp.int32)
    csum = jnp.cumsum(oh, axis=0)
    rank = jnp.take_along_axis(csum, e_flat[:, None], axis=1)[:, 0] - 1
    counts = csum[-1]
    tiles_e = (counts + TM - 1) // TM
    tile_start = jnp.concatenate(
        [jnp.zeros((1,), jnp.int32), jnp.cumsum(tiles_e)[:-1]])
    nt_live = jnp.sum(tiles_e)
    dest = tile_start[e_flat] * TM + rank
    src_row = jnp.zeros((NS,), jnp.int32).at[dest].set(
        jnp.arange(P, dtype=jnp.int32) // K)
    w_sorted = jnp.zeros((NS, 1), jnp.float32).at[dest, 0].set(
        topk_vals.reshape(-1))
    tid = jnp.arange(NT, dtype=jnp.int32)
    emap = jnp.sum(tid[:, None] >= tile_start[None, :], axis=1) - 1
    emap = jnp.clip(emap, 0, E - 1)
    last_live = jnp.max(jnp.where(tiles_e > 0, jnp.arange(E), -1))
    emap = jnp.where(tid < nt_live, emap, last_live).astype(jnp.int32)
    valid = (tid < nt_live).astype(jnp.int32)
    return src_row, w_sorted, emap, valid, dest


def _sc_gather(table, idx):
    b_per_w = NS // _NW
    bc = b_per_w // 2
    mesh = plsc.VectorSubcoreMesh(core_axis_name="c", subcore_axis_name="s")

    @functools.partial(
        pl.kernel, mesh=mesh,
        out_type=jax.ShapeDtypeStruct((NS, H), jnp.float32),
        scratch_types=[pltpu.VMEM((bc,), jnp.int32),
                       pltpu.VMEM((bc, H), jnp.float32),
                       pltpu.SemaphoreType.DMA])
    def k(table_hbm, idx_hbm, out_hbm, idx_v, rows_v, sem):
        wid = lax.axis_index("s") * 2 + lax.axis_index("c")

        @pl.loop(0, 2)
        def _(c):
            base = wid * b_per_w + c * bc
            pltpu.sync_copy(idx_hbm.at[pl.ds(base, bc)], idx_v)
            pltpu.async_copy(table_hbm.at[idx_v], rows_v, sem).wait()
            pltpu.sync_copy(rows_v, out_hbm.at[pl.ds(base, bc)])

    return k(table, idx)


def _ffn1(emap, valid, xs, w1, w3):
    def body(emap_ref, valid_ref, x_ref, w1_ref, w3_ref, a_ref):
        t = pl.program_id(1)

        @pl.when(valid_ref[t] != 0)
        def _():
            xb = x_ref[...]
            w1b = w1_ref[0]
            w3b = w3_ref[0]
            dims = (((1,), (1,)), ((), ()))
            g = lax.dot_general(xb, w1b, dims,
                                preferred_element_type=jnp.float32)
            u = lax.dot_general(xb, w3b, dims,
                                preferred_element_type=jnp.float32)
            a_ref[...] = (g * jax.nn.sigmoid(g)) * u

    grid_spec = pltpu.PrefetchScalarGridSpec(
        num_scalar_prefetch=2,
        grid=(NI, NT),
        in_specs=[
            pl.BlockSpec((TM, H), lambda i, t, em, vl: (t, 0)),
            pl.BlockSpec((1, TI, H), lambda i, t, em, vl: (em[t], i, 0)),
            pl.BlockSpec((1, TI, H), lambda i, t, em, vl: (em[t], i, 0)),
        ],
        out_specs=pl.BlockSpec((TM, TI), lambda i, t, em, vl: (t, i)),
    )
    return pl.pallas_call(
        body,
        grid_spec=grid_spec,
        out_shape=jax.ShapeDtypeStruct((NS, I), jnp.float32),
        compiler_params=pltpu.CompilerParams(
            dimension_semantics=("arbitrary", "arbitrary")),
    )(emap, valid, xs, w1, w3)


def _ffn2(emap, valid, a, w2, w_sorted):
    def body(emap_ref, valid_ref, a_ref, w2_ref, ws_ref, y_ref):
        t = pl.program_id(0)

        @pl.when(valid_ref[t] != 0)
        def _():
            ab = a_ref[...]
            w2b = w2_ref[0]
            dims = (((1,), (1,)), ((), ()))
            y = lax.dot_general(ab, w2b, dims,
                                preferred_element_type=jnp.float32)
            y_ref[...] = y * ws_ref[...]

    grid_spec = pltpu.PrefetchScalarGridSpec(
        num_scalar_prefetch=2,
        grid=(NT,),
        in_specs=[
            pl.BlockSpec((TM, I), lambda t, em, vl: (t, 0)),
            pl.BlockSpec((1, H, I), lambda t, em, vl: (em[t], 0, 0)),
            pl.BlockSpec((TM, 1), lambda t, em, vl: (t, 0)),
        ],
        out_specs=pl.BlockSpec((TM, H), lambda t, em, vl: (t, 0)),
    )
    return pl.pallas_call(
        body,
        grid_spec=grid_spec,
        out_shape=jax.ShapeDtypeStruct((NS, H), jnp.float32),
        compiler_params=pltpu.CompilerParams(
            dimension_semantics=("arbitrary",)),
    )(emap, valid, a, w2, w_sorted)


def _sc_combine(y, d0, d1):
    per_w = T // _NW
    bc = per_w // 2
    mesh = plsc.VectorSubcoreMesh(core_axis_name="c", subcore_axis_name="s")

    @functools.partial(
        pl.kernel, mesh=mesh,
        out_type=jax.ShapeDtypeStruct((T, H), jnp.float32),
        scratch_types=[pltpu.VMEM((bc,), jnp.int32),
                       pltpu.VMEM((bc,), jnp.int32),
                       pltpu.VMEM((bc, H), jnp.float32),
                       pltpu.VMEM((bc, H), jnp.float32),
                       pltpu.SemaphoreType.DMA])
    def k(y_hbm, d0_hbm, d1_hbm, out_hbm, i0_v, i1_v, r0_v, r1_v, sem):
        wid = lax.axis_index("s") * 2 + lax.axis_index("c")

        @pl.loop(0, 2)
        def _(c):
            base = wid * per_w + c * bc
            pltpu.sync_copy(d0_hbm.at[pl.ds(base, bc)], i0_v)
            pltpu.sync_copy(d1_hbm.at[pl.ds(base, bc)], i1_v)
            pltpu.async_copy(y_hbm.at[i0_v], r0_v, sem).wait()
            pltpu.async_copy(y_hbm.at[i1_v], r1_v, sem).wait()

            @pl.loop(0, bc)
            def _(r):
                @pl.loop(0, H, step=16)
                def _(j):
                    r0_v[r, pl.ds(j, 16)] = (r0_v[r, pl.ds(j, 16)]
                                             + r1_v[r, pl.ds(j, 16)])

            pltpu.sync_copy(r0_v, out_hbm.at[pl.ds(base, bc)])

    return k(y, d0, d1)


def kernel(hidden_states, gate_w, w1, w3, w2):
    orig_shape = hidden_states.shape
    x = hidden_states.reshape(T, H)
    gate_w_pad = jnp.zeros((H, EP), jnp.float32).at[:, :E].set(gate_w)

    topk_vals, topk_idx = _router(x, gate_w_pad)
    src_row, w_sorted, emap, valid, dest = _dispatch_meta(topk_vals, topk_idx)

    xs = _sc_gather(x, src_row)
    a = _ffn1(emap, valid, xs, w1, w3)
    y = _ffn2(emap, valid, a, w2, w_sorted)
    out = _sc_combine(y, dest[0::2], dest[1::2])
    return out.reshape(orig_shape)
